# baseline (speedup 1.0000x reference)
import jax
import jax.numpy as jnp
from jax import lax
from jax.experimental import pallas as pl
from jax.experimental.pallas import tpu as pltpu
from jax.experimental.pallas import tpu_sc as plsc

N = 100000
E = 1600000
M3 = 200000
G = 512
D = 128
H = 128

NC = 2
NS = 16
NW = NC * NS
L = 16

RB = 1024
N_SPLIT = 54272
NBLK = N_SPLIT // RB

CHUNK = 32
BASE_CH = N_SPLIT // CHUNK
N_SC = N - N_SPLIT
NCH = N_SC // CHUNK
CH_LO = NCH // NW
CH_EXTRA = NCH - CH_LO * NW
R = 5

EPW = E // NW
ECH = 2000
ECHN = EPW // ECH
E_START = 12
E_EVERY = 1
MCH = 2000
M_CHUNKS = M3 // MCH
M_MAX = -(-M_CHUNKS // NW)
M_START = E_START + E_EVERY * ECHN


def _sc_body(x_hbm, b_hbm, c2_hbm, c3_hbm,
             pooled_out, cnt_out, deg_out, mot_out,
             b_full, xb0, xb1, xb2, xb3, xb4,
             idx0, idx1, idx2, idx3, idx4,
             eb0, eb1, hist, hist2, hist3,
             pooled_sh, sem_b, sem_x0, sem_x1, sem_x2, sem_x3, sem_x4,
             sem_s0, sem_s1, sem_s2, sem_s3, sem_s4, sem_e0, sem_e1):
    cid = lax.axis_index("c")
    sid = lax.axis_index("s")
    wid = sid * NC + cid

    ones16 = jnp.ones((L,), jnp.float32)
    zeros16 = jnp.zeros((L,), jnp.float32)
    xb = (xb0, xb1, xb2, xb3, xb4)
    idx = (idx0, idx1, idx2, idx3, idx4)
    eb = (eb0, eb1)
    sem_x = (sem_x0, sem_x1, sem_x2, sem_x3, sem_x4)
    sem_s = (sem_s0, sem_s1, sem_s2, sem_s3, sem_s4)
    sem_e = (sem_e0, sem_e1)

    b_cp = pltpu.async_copy(b_hbm, b_full, sem_b)

    def zero(ref):
        def zbody(i, carry):
            ref[pl.ds(i * L, L)] = zeros16
            return carry
        lax.fori_loop(0, G // L, zbody, None)

    rows_per_tile = G // NS
    with jax.named_scope("ph0_zero"):
        zero(hist)
        zero(hist2)
        zero(hist3)

        def zrow(i, carry):
            def zcol(k, c2):
                xb0[i, pl.ds(k * L, L)] = zeros16
                return c2
            lax.fori_loop(0, D // L, zcol, None)
            return carry
        lax.fori_loop(0, rows_per_tile, zrow, None)
        pltpu.sync_copy(xb0.at[pl.ds(0, rows_per_tile)],
                        pooled_sh.at[pl.ds(sid * rows_per_tile,
                                           rows_per_tile)])
        plsc.subcore_barrier()

    c0 = wid * CH_LO + jnp.minimum(wid, CH_EXTRA)
    has_extra = wid < CH_EXTRA
    ebase = wid * EPW

    def load_cp(k, p):
        row0 = (BASE_CH + c0 + k) * CHUNK
        return (pltpu.make_async_copy(
                    x_hbm.at[pl.ds(row0, CHUNK)], xb[p], sem_x[p]),
                pltpu.make_async_copy(
                    b_hbm.at[pl.ds(row0, CHUNK)], idx[p], sem_x[p]))

    def load(k, p):
        a, b_ = load_cp(k, p)
        a.start()
        b_.start()

    def load_wait(k, p):
        a, b_ = load_cp(k, p)
        a.wait()
        b_.wait()

    def scat(k, p, fire):
        cp = pltpu.make_async_copy(xb[p], pooled_sh.at[idx[p]], sem_s[p])
        if fire:
            cp.start(add=True)
        else:
            cp.wait()

    def cnt_hist(p):
        for j in range(CHUNK // L):
            plsc.addupdate_scatter(hist, [idx[p][pl.ds(j * L, L)]], ones16)

    def e_load(m, q):
        pltpu.async_copy(c2_hbm.at[pl.ds(ebase + m * ECH, ECH)], eb[q],
                         sem_e[q])

    def e_chunk(m, q):
        pltpu.make_async_copy(c2_hbm.at[pl.ds(ebase + m * ECH, ECH)],
                              eb[q], sem_e[q]).wait()
        if m + 1 < ECHN:
            e_load(m + 1, 1 - q)

        @plsc.parallel_loop(0, ECH // L, unroll=16)
        def _g(i):
            nidx = eb[q][pl.ds(i * L, L)]
            g = plsc.load_gather(b_full, [nidx])
            plsc.addupdate_scatter(hist2, [g], ones16)

    def m_cp(m):
        ch = wid + m * NW
        q = (m + 1) % 2
        return pltpu.make_async_copy(c3_hbm.at[pl.ds(ch * MCH, MCH)],
                                     eb[q], sem_e[q]), q, ch

    def m_fire(m):
        cp, _, ch = m_cp(m)

        @pl.when(ch < M_CHUNKS)
        def _():
            cp.start()

    def m_chunk(m):
        cp, q, ch = m_cp(m)

        @pl.when(ch < M_CHUNKS)
        def _():
            cp.wait()

            @plsc.parallel_loop(0, MCH // L, unroll=16)
            def _g(j):
                nidx = eb[q][pl.ds(j * L, L)]
                g = plsc.load_gather(b_full, [nidx])
                plsc.addupdate_scatter(hist3, [g], ones16)

    with jax.named_scope("phAB_pool_deg"):
        e_load(0, 0)
        load(0, 0)
        load(1, 1)
        load(2, 2)
        for k in range(CH_LO):
            p = k % R
            load_wait(k, p)
            scat(k, p, fire=True)
            if k >= 2:
                scat(k - 2, (k - 2) % R, fire=False)
            if k + 3 < CH_LO:
                load(k + 3, (k + 3) % R)
            elif k + 3 == CH_LO:
                @pl.when(has_extra)
                def _():
                    load(CH_LO, CH_LO % R)
            cnt_hist(p)
            if k >= E_START and (k - E_START) % E_EVERY == 0:
                m = (k - E_START) // E_EVERY
                if m < ECHN:
                    if m == 0:
                        b_cp.wait()
                    e_chunk(m, m % 2)
            if k >= M_START - 1:
                m = k - (M_START - 1)
                if m < M_MAX:
                    m_fire(m)
            if k >= M_START:
                m = k - M_START
                if m < M_MAX:
                    m_chunk(m)

        @pl.when(has_extra)
        def _extra():
            p = CH_LO % R
            load_wait(CH_LO, p)
            scat(CH_LO, p, fire=True)
            cnt_hist(p)
            scat(CH_LO, p, fire=False)
        scat(CH_LO - 2, (CH_LO - 2) % R, fire=False)
        scat(CH_LO - 1, (CH_LO - 1) % R, fire=False)

        pltpu.sync_copy(hist, cnt_out.at[wid])
        pltpu.sync_copy(hist2, deg_out.at[wid])
        pltpu.sync_copy(hist3, mot_out.at[wid])
        plsc.subcore_barrier()

    with jax.named_scope("phW_writeout"):
        gbase = cid * G + sid * rows_per_tile
        pltpu.sync_copy(pooled_sh.at[pl.ds(sid * rows_per_tile,
                                           rows_per_tile)],
                        xb0.at[pl.ds(0, rows_per_tile)])
        pltpu.sync_copy(xb0.at[pl.ds(0, rows_per_tile)],
                        pooled_out.at[pl.ds(gbase, rows_per_tile)])


_sc_kernel = pl.kernel(
    _sc_body,
    out_type=[
        jax.ShapeDtypeStruct((NC * G, D), jnp.float32),
        jax.ShapeDtypeStruct((NW, G), jnp.float32),
        jax.ShapeDtypeStruct((NW, G), jnp.float32),
        jax.ShapeDtypeStruct((NW, G), jnp.float32),
    ],
    mesh=plsc.VectorSubcoreMesh(core_axis_name="c", subcore_axis_name="s"),
    scratch_types=[
        pltpu.VMEM((N,), jnp.int32),
        pltpu.VMEM((CHUNK, D), jnp.float32),
        pltpu.VMEM((CHUNK, D), jnp.float32),
        pltpu.VMEM((CHUNK, D), jnp.float32),
        pltpu.VMEM((CHUNK, D), jnp.float32),
        pltpu.VMEM((CHUNK, D), jnp.float32),
        pltpu.VMEM((CHUNK,), jnp.int32),
        pltpu.VMEM((CHUNK,), jnp.int32),
        pltpu.VMEM((CHUNK,), jnp.int32),
        pltpu.VMEM((CHUNK,), jnp.int32),
        pltpu.VMEM((CHUNK,), jnp.int32),
        pltpu.VMEM((ECH,), jnp.int32),
        pltpu.VMEM((ECH,), jnp.int32),
        pltpu.VMEM((G,), jnp.float32),
        pltpu.VMEM((G,), jnp.float32),
        pltpu.VMEM((G,), jnp.float32),
        pltpu.VMEM_SHARED((G, D), jnp.float32),
        pltpu.SemaphoreType.DMA,
        pltpu.SemaphoreType.DMA,
        pltpu.SemaphoreType.DMA,
        pltpu.SemaphoreType.DMA,
        pltpu.SemaphoreType.DMA,
        pltpu.SemaphoreType.DMA,
        pltpu.SemaphoreType.DMA,
        pltpu.SemaphoreType.DMA,
        pltpu.SemaphoreType.DMA,
        pltpu.SemaphoreType.DMA,
        pltpu.SemaphoreType.DMA,
        pltpu.SemaphoreType.DMA,
        pltpu.SemaphoreType.DMA,
    ],
    compiler_params=pltpu.CompilerParams(needs_layout_passes=False),
    name="graph_stats_sc",
)


def _pool_tc_body(x_ref, b_ref, out_ref, cnt_ref):
    i = pl.program_id(0)
    hit = lax.broadcasted_iota(jnp.int32, (G, RB), 0) == b_ref[0]
    onehot = hit.astype(jnp.float32)
    part = jnp.dot(onehot, x_ref[...], preferred_element_type=jnp.float32)
    pcnt = jnp.sum(onehot, axis=1, keepdims=True)

    @pl.when(i == 0)
    def _():
        out_ref[...] = jnp.zeros_like(out_ref)
        cnt_ref[...] = jnp.zeros_like(cnt_ref)
    out_ref[...] += part
    cnt_ref[...] += pcnt


_pool_tc_kernel = pl.pallas_call(
    _pool_tc_body,
    grid=(NBLK,),
    in_specs=[
        pl.BlockSpec((RB, D), lambda i: (i, 0)),
        pl.BlockSpec((1, 1, RB), lambda i: (i, 0, 0)),
    ],
    out_specs=[
        pl.BlockSpec((G, D), lambda i: (0, 0)),
        pl.BlockSpec((G, 1), lambda i: (0, 0)),
    ],
    out_shape=[
        jax.ShapeDtypeStruct((G, D), jnp.float32),
        jax.ShapeDtypeStruct((G, 1), jnp.float32),
    ],
)


def _tc_body(pp, ptc, ctc, cp, dp, mp, W1_ref, b1_ref, W2, b2_ref, w3,
             b3_ref, out_ref):
    pooled = pp[pl.ds(0, G), :] + pp[pl.ds(G, G), :] + ptc[...]
    counts = jnp.maximum(jnp.sum(cp[...], axis=0) + ctc[...][:, 0], 1.0)
    deg = jnp.sum(dp[...], axis=0)
    mot = jnp.sum(mp[...], axis=0)
    inv = 1.0 / counts
    mean_x = pooled * inv[:, None]
    W1a = W1_ref[pl.ds(0, D), :]
    w1d = W1_ref[pl.ds(D, 1), :]
    w1m = W1_ref[pl.ds(D + 1, 1), :]
    pre1 = jnp.dot(mean_x, W1a, preferred_element_type=jnp.float32,
                   precision=lax.Precision.HIGHEST)
    pre1 = (pre1 + (deg * inv)[:, None] * w1d
            + (mot * inv)[:, None] * w1m + b1_ref[...][None, :])
    h1 = jnp.maximum(pre1, 0.0)
    h2 = jnp.maximum(
        jnp.dot(h1, W2[...], preferred_element_type=jnp.float32,
                precision=lax.Precision.HIGHEST)
        + b2_ref[...][None, :], 0.0)
    out2 = jnp.dot(h2, w3[...], preferred_element_type=jnp.float32,
                   precision=lax.Precision.HIGHEST)
    out_ref[...] = out2[:, 0] + b3_ref[0]


_tc_kernel = pl.pallas_call(
    _tc_body,
    out_shape=jax.ShapeDtypeStruct((G,), jnp.float32),
)


def kernel(x, b, c_2, c_3, num_graphs, W1, b1, W2, b2, W3, b3):
    del num_graphs
    b3d = b[:N_SPLIT].reshape(NBLK, 1, RB)
    pooled_p, cnt_p, deg_p, mot_p = _sc_kernel(x, b, c_2, c_3)
    pooled_tc, cnt_tc = _pool_tc_kernel(x, b3d)
    return _tc_kernel(pooled_p, pooled_tc, cnt_tc, cnt_p, deg_p, mot_p,
                      W1, b1, W2, b2, W3, b3)


# speedup vs baseline: 1.0531x; 1.0531x over previous
import jax
import jax.numpy as jnp
from jax import lax
from jax.experimental import pallas as pl
from jax.experimental.pallas import tpu as pltpu
from jax.experimental.pallas import tpu_sc as plsc

N = 100000
E = 1600000
M3 = 200000
G = 512
D = 128
H = 128

NC = 2
NS = 16
NW = NC * NS
L = 16

RB = 1024
N_SPLIT = 39936
NBLK = N_SPLIT // RB

CHUNK = 32
BASE_CH = N_SPLIT // CHUNK
N_SC = N - N_SPLIT
NCH = N_SC // CHUNK
CH_LO = NCH // NW
CH_EXTRA = NCH - CH_LO * NW
R = 5

EPW = E // NW
ECH = 2000
ECHN = EPW // ECH
E_START = 8
E_EVERY = 2
MCH = 2000
M_CHUNKS = M3 // MCH
M_MAX = -(-M_CHUNKS // NW)


def _sc_body(x_hbm, b_hbm, c2_hbm, c3_hbm,
             pooled_out, cnt_out, deg_out, mot_out,
             b_full, xb0, xb1, xb2, xb3, xb4,
             idx0, idx1, idx2, idx3, idx4,
             eb0, eb1, hist, hist2, hist3,
             pooled_sh, sem_b, sem_x0, sem_x1, sem_x2, sem_x3, sem_x4,
             sem_s0, sem_s1, sem_s2, sem_s3, sem_s4, sem_e0, sem_e1):
    cid = lax.axis_index("c")
    sid = lax.axis_index("s")
    wid = sid * NC + cid

    ones16 = jnp.ones((L,), jnp.float32)
    zeros16 = jnp.zeros((L,), jnp.float32)
    xb = (xb0, xb1, xb2, xb3, xb4)
    idx = (idx0, idx1, idx2, idx3, idx4)
    eb = (eb0, eb1)
    sem_x = (sem_x0, sem_x1, sem_x2, sem_x3, sem_x4)
    sem_s = (sem_s0, sem_s1, sem_s2, sem_s3, sem_s4)
    sem_e = (sem_e0, sem_e1)

    b_cp = pltpu.async_copy(b_hbm, b_full, sem_b)

    def zero(ref):
        def zbody(i, carry):
            ref[pl.ds(i * L, L)] = zeros16
            return carry
        lax.fori_loop(0, G // L, zbody, None)

    rows_per_tile = G // NS
    with jax.named_scope("ph0_zero"):
        zero(hist)
        zero(hist2)
        zero(hist3)

        def zrow(i, carry):
            def zcol(k, c2):
                xb0[i, pl.ds(k * L, L)] = zeros16
                return c2
            lax.fori_loop(0, D // L, zcol, None)
            return carry
        lax.fori_loop(0, rows_per_tile, zrow, None)
        pltpu.sync_copy(xb0.at[pl.ds(0, rows_per_tile)],
                        pooled_sh.at[pl.ds(sid * rows_per_tile,
                                           rows_per_tile)])
        plsc.subcore_barrier()

    c0 = wid * CH_LO + jnp.minimum(wid, CH_EXTRA)
    has_extra = wid < CH_EXTRA
    ebase = wid * EPW

    def load_cp(k, p):
        row0 = (BASE_CH + c0 + k) * CHUNK
        return (pltpu.make_async_copy(
                    x_hbm.at[pl.ds(row0, CHUNK)], xb[p], sem_x[p]),
                pltpu.make_async_copy(
                    b_hbm.at[pl.ds(row0, CHUNK)], idx[p], sem_x[p]))

    def load(k, p):
        a, b_ = load_cp(k, p)
        a.start()
        b_.start()

    def load_wait(k, p):
        a, b_ = load_cp(k, p)
        a.wait()
        b_.wait()

    def scat(k, p, fire):
        cp = pltpu.make_async_copy(xb[p], pooled_sh.at[idx[p]], sem_s[p])
        if fire:
            cp.start(add=True)
        else:
            cp.wait()

    def cnt_hist(p):
        for j in range(CHUNK // L):
            plsc.addupdate_scatter(hist, [idx[p][pl.ds(j * L, L)]], ones16)

    def e_load(m, q):
        pltpu.async_copy(c2_hbm.at[pl.ds(ebase + m * ECH, ECH)], eb[q],
                         sem_e[q])

    def e_chunk(m, q):
        pltpu.make_async_copy(c2_hbm.at[pl.ds(ebase + m * ECH, ECH)],
                              eb[q], sem_e[q]).wait()
        if m + 1 < ECHN:
            e_load(m + 1, 1 - q)

        @plsc.parallel_loop(0, ECH // L, unroll=16)
        def _g(i):
            nidx = eb[q][pl.ds(i * L, L)]
            g = plsc.load_gather(b_full, [nidx])
            plsc.addupdate_scatter(hist2, [g], ones16)

    def m_cp(m):
        ch = wid + m * NW
        q = (m + 1) % 2
        return pltpu.make_async_copy(c3_hbm.at[pl.ds(ch * MCH, MCH)],
                                     eb[q], sem_e[q]), q, ch

    def m_fire(m):
        cp, _, ch = m_cp(m)

        @pl.when(ch < M_CHUNKS)
        def _():
            cp.start()

    def m_chunk(m):
        cp, q, ch = m_cp(m)

        @pl.when(ch < M_CHUNKS)
        def _():
            cp.wait()

            @plsc.parallel_loop(0, MCH // L, unroll=16)
            def _g(j):
                nidx = eb[q][pl.ds(j * L, L)]
                g = plsc.load_gather(b_full, [nidx])
                plsc.addupdate_scatter(hist3, [g], ones16)

    with jax.named_scope("phAB_pool_deg"):
        e_load(0, 0)
        load(0, 0)
        load(1, 1)
        load(2, 2)
        for k in range(CH_LO):
            p = k % R
            load_wait(k, p)
            scat(k, p, fire=True)
            if k >= 2:
                scat(k - 2, (k - 2) % R, fire=False)
            if k + 3 < CH_LO:
                load(k + 3, (k + 3) % R)
            elif k + 3 == CH_LO:
                @pl.when(has_extra)
                def _():
                    load(CH_LO, CH_LO % R)
            cnt_hist(p)
            if k >= E_START and (k - E_START) % E_EVERY == 0:
                m = (k - E_START) // E_EVERY
                if m < ECHN:
                    if m == 0:
                        b_cp.wait()
                    e_chunk(m, m % 2)

        @pl.when(has_extra)
        def _extra():
            p = CH_LO % R
            load_wait(CH_LO, p)
            scat(CH_LO, p, fire=True)
            cnt_hist(p)
            scat(CH_LO, p, fire=False)
        m_fire(0)
        scat(CH_LO - 2, (CH_LO - 2) % R, fire=False)
        scat(CH_LO - 1, (CH_LO - 1) % R, fire=False)
        for m in range(M_MAX):
            if m + 1 < M_MAX:
                m_fire(m + 1)
            m_chunk(m)

        pltpu.sync_copy(hist, cnt_out.at[wid])
        pltpu.sync_copy(hist2, deg_out.at[wid])
        pltpu.sync_copy(hist3, mot_out.at[wid])
        plsc.subcore_barrier()

    with jax.named_scope("phW_writeout"):
        gbase = cid * G + sid * rows_per_tile
        pltpu.sync_copy(pooled_sh.at[pl.ds(sid * rows_per_tile,
                                           rows_per_tile)],
                        xb0.at[pl.ds(0, rows_per_tile)])
        pltpu.sync_copy(xb0.at[pl.ds(0, rows_per_tile)],
                        pooled_out.at[pl.ds(gbase, rows_per_tile)])


_sc_kernel = pl.kernel(
    _sc_body,
    out_type=[
        jax.ShapeDtypeStruct((NC * G, D), jnp.float32),
        jax.ShapeDtypeStruct((NW, G), jnp.float32),
        jax.ShapeDtypeStruct((NW, G), jnp.float32),
        jax.ShapeDtypeStruct((NW, G), jnp.float32),
    ],
    mesh=plsc.VectorSubcoreMesh(core_axis_name="c", subcore_axis_name="s"),
    scratch_types=[
        pltpu.VMEM((N,), jnp.int32),
        pltpu.VMEM((CHUNK, D), jnp.float32),
        pltpu.VMEM((CHUNK, D), jnp.float32),
        pltpu.VMEM((CHUNK, D), jnp.float32),
        pltpu.VMEM((CHUNK, D), jnp.float32),
        pltpu.VMEM((CHUNK, D), jnp.float32),
        pltpu.VMEM((CHUNK,), jnp.int32),
        pltpu.VMEM((CHUNK,), jnp.int32),
        pltpu.VMEM((CHUNK,), jnp.int32),
        pltpu.VMEM((CHUNK,), jnp.int32),
        pltpu.VMEM((CHUNK,), jnp.int32),
        pltpu.VMEM((ECH,), jnp.int32),
        pltpu.VMEM((ECH,), jnp.int32),
        pltpu.VMEM((G,), jnp.float32),
        pltpu.VMEM((G,), jnp.float32),
        pltpu.VMEM((G,), jnp.float32),
        pltpu.VMEM_SHARED((G, D), jnp.float32),
        pltpu.SemaphoreType.DMA,
        pltpu.SemaphoreType.DMA,
        pltpu.SemaphoreType.DMA,
        pltpu.SemaphoreType.DMA,
        pltpu.SemaphoreType.DMA,
        pltpu.SemaphoreType.DMA,
        pltpu.SemaphoreType.DMA,
        pltpu.SemaphoreType.DMA,
        pltpu.SemaphoreType.DMA,
        pltpu.SemaphoreType.DMA,
        pltpu.SemaphoreType.DMA,
        pltpu.SemaphoreType.DMA,
        pltpu.SemaphoreType.DMA,
    ],
    compiler_params=pltpu.CompilerParams(needs_layout_passes=False),
    name="graph_stats_sc",
)


def _pool_tc_body(x_ref, b_ref, out_ref, cnt_ref):
    i = pl.program_id(0)
    hit = lax.broadcasted_iota(jnp.int32, (G, RB), 0) == b_ref[0]
    onehot = hit.astype(jnp.float32)
    part = jnp.dot(onehot, x_ref[...], preferred_element_type=jnp.float32)
    pcnt = jnp.sum(onehot, axis=1, keepdims=True)

    @pl.when(i == 0)
    def _():
        out_ref[...] = jnp.zeros_like(out_ref)
        cnt_ref[...] = jnp.zeros_like(cnt_ref)
    out_ref[...] += part
    cnt_ref[...] += pcnt


_pool_tc_kernel = pl.pallas_call(
    _pool_tc_body,
    grid=(NBLK,),
    in_specs=[
        pl.BlockSpec((RB, D), lambda i: (i, 0)),
        pl.BlockSpec((1, 1, RB), lambda i: (i, 0, 0)),
    ],
    out_specs=[
        pl.BlockSpec((G, D), lambda i: (0, 0)),
        pl.BlockSpec((G, 1), lambda i: (0, 0)),
    ],
    out_shape=[
        jax.ShapeDtypeStruct((G, D), jnp.float32),
        jax.ShapeDtypeStruct((G, 1), jnp.float32),
    ],
)


def _tc_body(pp, ptc, ctc, cp, dp, mp, W1_ref, b1_ref, W2, b2_ref, w3,
             b3_ref, out_ref):
    pooled = pp[pl.ds(0, G), :] + pp[pl.ds(G, G), :] + ptc[...]
    counts = jnp.maximum(jnp.sum(cp[...], axis=0) + ctc[...][:, 0], 1.0)
    deg = jnp.sum(dp[...], axis=0)
    mot = jnp.sum(mp[...], axis=0)
    inv = 1.0 / counts
    mean_x = pooled * inv[:, None]
    W1a = W1_ref[pl.ds(0, D), :]
    w1d = W1_ref[pl.ds(D, 1), :]
    w1m = W1_ref[pl.ds(D + 1, 1), :]
    pre1 = jnp.dot(mean_x, W1a, preferred_element_type=jnp.float32,
                   precision=lax.Precision.HIGHEST)
    pre1 = (pre1 + (deg * inv)[:, None] * w1d
            + (mot * inv)[:, None] * w1m + b1_ref[...][None, :])
    h1 = jnp.maximum(pre1, 0.0)
    h2 = jnp.maximum(
        jnp.dot(h1, W2[...], preferred_element_type=jnp.float32,
                precision=lax.Precision.HIGHEST)
        + b2_ref[...][None, :], 0.0)
    out2 = jnp.dot(h2, w3[...], preferred_element_type=jnp.float32,
                   precision=lax.Precision.HIGHEST)
    out_ref[...] = out2[:, 0] + b3_ref[0]


_tc_kernel = pl.pallas_call(
    _tc_body,
    out_shape=jax.ShapeDtypeStruct((G,), jnp.float32),
)


def kernel(x, b, c_2, c_3, num_graphs, W1, b1, W2, b2, W3, b3):
    del num_graphs
    b3d = b[:N_SPLIT].reshape(NBLK, 1, RB)
    pooled_p, cnt_p, deg_p, mot_p = _sc_kernel(x, b, c_2, c_3)
    pooled_tc, cnt_tc = _pool_tc_kernel(x, b3d)
    return _tc_kernel(pooled_p, pooled_tc, cnt_tc, cnt_p, deg_p, mot_p,
                      W1, b1, W2, b2, W3, b3)
